# baseline (speedup 1.0000x reference)
import functools
import jax
import jax.numpy as jnp
from jax import lax
from jax.experimental import pallas as pl
from jax.experimental.pallas import tpu as pltpu
from jax.experimental.pallas import tpu_sc as plsc

N = 10000
NNZ = 320000
NUM_GRAPHS = 64
NPAD = 10240
CBLK = 512
NCB = NPAD // CBLK
EC = 8000
EPS = 1e-16


def _proj0_body(x_ref, w_ref, a_ref, h_ref, s_ref):
    h = jax.lax.dot(w_ref[...], x_ref[...], preferred_element_type=jnp.float32)
    h_ref[...] = h
    s_ref[...] = jax.lax.dot(a_ref[...], h, preferred_element_type=jnp.float32)


def _combine(nums, dens):
    nu = nums[0, 1] + nums[1, 1]
    nd = nums[0, 2] + nums[1, 2]
    np_ = nums[0, 0] + nums[1, 0]
    du = (dens[0] + dens[2])[None, :]
    dd = (dens[1] + dens[3])[None, :]
    return jax.nn.relu(nu / (du + EPS) + nd / (dd + EPS) + np_)


def _projc_body(n_ref, d_ref, w_ref, a_ref, h_ref, s_ref):
    x = _combine(n_ref[...], d_ref[...])
    h = jax.lax.dot(w_ref[...], x, preferred_element_type=jnp.float32)
    h_ref[...] = h
    s_ref[...] = jax.lax.dot(a_ref[...], h, preferred_element_type=jnp.float32)


def _pool_body(n_ref, d_ref, b_ref, o_ref, acc_ref, cnt_ref):
    step = pl.program_id(0)

    @pl.when(step == 0)
    def _init():
        acc_ref[...] = jnp.zeros_like(acc_ref)
        cnt_ref[...] = jnp.zeros_like(cnt_ref)

    x = _combine(n_ref[...], d_ref[...])
    bids = b_ref[...].reshape(1, CBLK)
    gids = jax.lax.broadcasted_iota(jnp.int32, (NUM_GRAPHS, CBLK), 0)
    mask = (gids == bids).astype(jnp.float32)
    acc_ref[...] += jax.lax.dot_general(
        x, mask, (((1,), (1,)), ((), ())), preferred_element_type=jnp.float32)
    cnt_ref[:1, :] += jnp.sum(mask, axis=1)[None, :]

    @pl.when(step == NCB - 1)
    def _fin():
        mean = acc_ref[...] / jnp.maximum(cnt_ref[:1, :], 1.0)
        row = jax.lax.broadcasted_iota(jnp.int32, (32, NUM_GRAPHS), 0)
        z = jnp.where(row < 10, mean, -jnp.inf)
        m = jnp.max(z, axis=0, keepdims=True)
        e = jnp.exp(z - m)
        sm = e / jnp.sum(e, axis=0, keepdims=True)
        o_ref[...] = jnp.concatenate(
            [sm, jnp.zeros((32, 128 - NUM_GRAPHS), jnp.float32)], axis=1)


def _proj0(x1t, wcat_t, amat):
    return pl.pallas_call(
        _proj0_body,
        grid=(NCB,),
        in_specs=[
            pl.BlockSpec((128, CBLK), lambda i: (0, i)),
            pl.BlockSpec((96, 128), lambda i: (0, 0)),
            pl.BlockSpec((8, 96), lambda i: (0, 0)),
        ],
        out_specs=[
            pl.BlockSpec((96, CBLK), lambda i: (0, i)),
            pl.BlockSpec((8, CBLK), lambda i: (0, i)),
        ],
        out_shape=[
            jax.ShapeDtypeStruct((96, NPAD), jnp.float32),
            jax.ShapeDtypeStruct((8, NPAD), jnp.float32),
        ],
    )(x1t, wcat_t, amat)


def _projc(nums, dens, wcat_t, amat):
    return pl.pallas_call(
        _projc_body,
        grid=(NCB,),
        in_specs=[
            pl.BlockSpec((2, 3, 32, CBLK), lambda i: (0, 0, 0, i)),
            pl.BlockSpec((8, CBLK), lambda i: (0, i)),
            pl.BlockSpec((96, 32), lambda i: (0, 0)),
            pl.BlockSpec((8, 96), lambda i: (0, 0)),
        ],
        out_specs=[
            pl.BlockSpec((96, CBLK), lambda i: (0, i)),
            pl.BlockSpec((8, CBLK), lambda i: (0, i)),
        ],
        out_shape=[
            jax.ShapeDtypeStruct((96, NPAD), jnp.float32),
            jax.ShapeDtypeStruct((8, NPAD), jnp.float32),
        ],
    )(nums, dens, wcat_t, amat)


def _pool(nums, dens, batchp):
    return pl.pallas_call(
        _pool_body,
        grid=(NCB,),
        in_specs=[
            pl.BlockSpec((2, 3, 32, CBLK), lambda i: (0, 0, 0, i)),
            pl.BlockSpec((8, CBLK), lambda i: (0, i)),
            pl.BlockSpec((1, 1, CBLK), lambda i: (i, 0, 0)),
        ],
        out_specs=pl.BlockSpec((32, 128), lambda i: (0, 0)),
        out_shape=jax.ShapeDtypeStruct((32, 128), jnp.float32),
        scratch_shapes=[
            pltpu.VMEM((32, NUM_GRAPHS), jnp.float32),
            pltpu.VMEM((8, NUM_GRAPHS), jnp.float32),
        ],
    )(nums, dens, batchp)


_MESH = plsc.VectorSubcoreMesh(core_axis_name="c", subcore_axis_name="s")
HALF = NNZ // 2
PTILE = HALF // 16
SEG = NPAD // 16


def _zero(ref, base, n):
    @plsc.parallel_loop(0, n // 16, unroll=8)
    def body(k):
        ref[pl.ds(base + k * 16, 16)] = jnp.zeros((16,), jnp.float32)


def _unpack(pk):
    return jnp.right_shift(pk, 14), jnp.bitwise_and(pk, 16383)


def _sc_edges_body(last, htf, smf, l1pk, l1v, lupk, ldpk, nums, dens,
                   hbuf0, hbuf1, acc0, acc1, pbuf, wbuf, pbuf2, wbuf2,
                   mbuf, w_sh, slab, sem_a, sem_b):
    c = lax.axis_index("c")
    s = lax.axis_index("s")

    for g, epk in ((0, lupk), (1, ldpk)):
        pltpu.sync_copy(smf.at[pl.ds((2 * g) * NPAD, NPAD)], hbuf0)
        pltpu.sync_copy(smf.at[pl.ds((2 * g + 1) * NPAD, NPAD)], hbuf1)
        _zero(acc0, 0, NPAD)
        base0 = c * HALF + s * PTILE

        for cbase, clen in ((0, EC), (EC, PTILE - EC)):
            pltpu.sync_copy(epk.at[pl.ds(base0 + cbase, clen)],
                            pbuf.at[pl.ds(0, clen)])

            @plsc.parallel_loop(0, clen // 16, unroll=5)
            def grp0(q):
                off = q * 16
                i16, j16 = _unpack(pbuf[pl.ds(off, 16)])
                sd = plsc.load_gather(hbuf0, [i16])
                ss = plsc.load_gather(hbuf1, [j16])
                e = sd + ss
                e = jnp.maximum(e, 0.2 * e)
                w = jnp.exp(e)
                wbuf[pl.ds(off, 16)] = w
                plsc.addupdate_scatter(acc0, [i16], w)
            pltpu.sync_copy(
                wbuf.at[pl.ds(0, clen)],
                w_sh.at[pl.ds(g * NNZ + base0 + cbase, clen)])

        pltpu.sync_copy(acc0, slab.at[s])
        plsc.subcore_barrier()

        pltpu.sync_copy(slab.at[:, pl.ds(s * SEG, SEG)], mbuf)

        @plsc.parallel_loop(0, SEG // 16, unroll=2)
        def red(q):
            v = mbuf[0, pl.ds(q * 16, 16)]
            for t in range(1, 16):
                v = v + mbuf[t, pl.ds(q * 16, 16)]
            acc1[pl.ds(q * 16, 16)] = v
        pltpu.sync_copy(acc1.at[pl.ds(0, SEG)],
                        dens.at[pl.ds((2 * c + g) * NPAD + s * SEG, SEG)])
        if g == 0:
            _zero(acc0, 0, SEG)
            for r in range(4, 8):
                pltpu.sync_copy(acc0.at[pl.ds(0, SEG)],
                                dens.at[pl.ds(r * NPAD + s * SEG, SEG)])
        plsc.subcore_barrier()

    nchunk = HALF // EC

    def _phase(m, epk):
        f0 = 32 * m + 2 * s
        pltpu.sync_copy(htf.at[pl.ds(f0 * NPAD, NPAD)], hbuf0)
        pltpu.sync_copy(htf.at[pl.ds((f0 + 1) * NPAD, NPAD)], hbuf1)
        _zero(acc0, 0, NPAD)
        _zero(acc1, 0, NPAD)
        ebase = c * HALF

        def start_w(kc, wb, sem):
            if m == 0:
                return pltpu.async_copy(
                    l1v.at[pl.ds(ebase + kc * EC, EC)], wb, sem)
            return pltpu.async_copy(
                w_sh.at[pl.ds((m - 1) * NNZ + ebase + kc * EC, EC)], wb, sem)

        def compute(pb, wb):
            @plsc.parallel_loop(0, EC // 16, unroll=8)
            def grp(q):
                off = q * 16
                i16, j16 = _unpack(pb[pl.ds(off, 16)])
                w16 = wb[pl.ds(off, 16)]
                h0 = plsc.load_gather(hbuf0, [j16])
                plsc.addupdate_scatter(acc0, [i16], w16 * h0)
                h1 = plsc.load_gather(hbuf1, [j16])
                plsc.addupdate_scatter(acc1, [i16], w16 * h1)

        def pair(k, _):
            ha = pltpu.async_copy(
                epk.at[pl.ds(ebase + 2 * k * EC, EC)], pbuf, sem_a)
            hwa = start_w(2 * k, wbuf, sem_a)
            hb = pltpu.async_copy(
                epk.at[pl.ds(ebase + (2 * k + 1) * EC, EC)], pbuf2, sem_b)
            hwb = start_w(2 * k + 1, wbuf2, sem_b)
            ha.wait()
            hwa.wait()
            compute(pbuf, wbuf)
            hb.wait()
            hwb.wait()
            compute(pbuf2, wbuf2)
            return 0
        lax.fori_loop(0, nchunk // 2, pair, 0)

        row = (c * 3 + m) * 32 + 2 * s
        pltpu.sync_copy(acc0, nums.at[pl.ds(row * NPAD, NPAD)])
        pltpu.sync_copy(acc1, nums.at[pl.ds((row + 1) * NPAD, NPAD)])

    for m, epk in ((0, l1pk), (1, lupk), (2, ldpk)):
        if last:
            @pl.when(s < 5)
            def _active():
                _phase(m, epk)
        else:
            _phase(m, epk)


def _make_sc(last):
    return functools.partial(
        pl.kernel,
        mesh=_MESH,
        compiler_params=pltpu.CompilerParams(needs_layout_passes=False),
        out_type=[
            jax.ShapeDtypeStruct((2 * 3 * 32 * NPAD,), jnp.float32),
            jax.ShapeDtypeStruct((8 * NPAD,), jnp.float32),
        ],
        scratch_types=[
            pltpu.VMEM((NPAD,), jnp.float32),
            pltpu.VMEM((NPAD,), jnp.float32),
            pltpu.VMEM((NPAD,), jnp.float32),
            pltpu.VMEM((NPAD,), jnp.float32),
            pltpu.VMEM((EC,), jnp.int32),
            pltpu.VMEM((EC,), jnp.float32),
            pltpu.VMEM((EC,), jnp.int32),
            pltpu.VMEM((EC,), jnp.float32),
            pltpu.VMEM((16, SEG), jnp.float32),
            pltpu.HBM((2 * NNZ,), jnp.float32),
            pltpu.VMEM_SHARED((16, NPAD), jnp.float32),
            pltpu.SemaphoreType.DMA,
            pltpu.SemaphoreType.DMA,
        ],
    )(functools.partial(_sc_edges_body, last))


_sc_edges = _make_sc(False)
_sc_edges_last = _make_sc(True)


def _prep_params(params):
    out = []
    for p in params:
        din = p['Wp'].shape[0]
        dout = p['Wp'].shape[1]
        wp = jnp.zeros((din, 32), jnp.float32).at[:, :dout].set(p['Wp'])
        wu = jnp.zeros((din, 32), jnp.float32).at[:, :dout].set(p['Wu'])
        wd = jnp.zeros((din, 32), jnp.float32).at[:, :dout].set(p['Wd'])
        wcat_t = jnp.concatenate([wp.T, wu.T, wd.T], axis=0)
        amat = jnp.zeros((8, 96), jnp.float32)
        amat = amat.at[0, 32:32 + dout].set(p['au_d'])
        amat = amat.at[1, 32:32 + dout].set(p['au_s'])
        amat = amat.at[2, 64:64 + dout].set(p['ad_d'])
        amat = amat.at[3, 64:64 + dout].set(p['ad_s'])
        out.append((wcat_t, amat))
    return out


def kernel(X1, L1_idx, L1_val, Lu_idx, Lu_val, Ld_idx, Ld_val, batch1, params):
    pp = _prep_params(params)
    x1t = jnp.zeros((128, NPAD), jnp.float32).at[:, :N].set(X1.T)
    batchp = jnp.full((NPAD,), NUM_GRAPHS + 1, jnp.int32).at[:N].set(batch1)
    batchp = batchp.reshape(NCB, 1, CBLK)
    l1pk = jnp.left_shift(L1_idx[0], 14) | L1_idx[1]
    lupk = jnp.left_shift(Lu_idx[0], 14) | Lu_idx[1]
    ldpk = jnp.left_shift(Ld_idx[0], 14) | Ld_idx[1]

    ht, smat = _proj0(x1t, pp[0][0], pp[0][1])
    for l in range(1, 5):
        sc = _sc_edges_last if l == 4 else _sc_edges
        numsf, densf = sc(ht.reshape(-1), smat.reshape(-1),
                          l1pk, L1_val, lupk, ldpk)
        nums = numsf.reshape(2, 3, 32, NPAD)
        dens = densf.reshape(8, NPAD)
        if l < 4:
            ht, smat = _projc(nums, dens, pp[l][0], pp[l][1])
    out = _pool(nums, dens, batchp)
    return out[:10, :NUM_GRAPHS].T


# speedup vs baseline: 1.0163x; 1.0163x over previous
import functools
import jax
import jax.numpy as jnp
from jax import lax
from jax.experimental import pallas as pl
from jax.experimental.pallas import tpu as pltpu
from jax.experimental.pallas import tpu_sc as plsc

N = 10000
NNZ = 320000
NUM_GRAPHS = 64
NPAD = 10240
CBLK = 512
NCB = NPAD // CBLK
EC = 8000
EPS = 1e-16


def _proj0_body(x_ref, w_ref, a_ref, h_ref, s_ref):
    h = jax.lax.dot(w_ref[...], x_ref[...], preferred_element_type=jnp.float32)
    h_ref[...] = h
    s_ref[...] = jax.lax.dot(a_ref[...], h, preferred_element_type=jnp.float32)


def _combine(nums, dens):
    nu = nums[0, 1] + nums[1, 1]
    nd = nums[0, 2] + nums[1, 2]
    np_ = nums[0, 0] + nums[1, 0]
    du = (dens[0] + dens[2])[None, :]
    dd = (dens[1] + dens[3])[None, :]
    return jax.nn.relu(nu / (du + EPS) + nd / (dd + EPS) + np_)


def _projc_body(n_ref, d_ref, w_ref, a_ref, h_ref, s_ref):
    x = _combine(n_ref[...], d_ref[...])
    h = jax.lax.dot(w_ref[...], x, preferred_element_type=jnp.float32)
    h_ref[...] = h
    s_ref[...] = jax.lax.dot(a_ref[...], h, preferred_element_type=jnp.float32)


def _pool_body(n_ref, d_ref, b_ref, o_ref, acc_ref, cnt_ref):
    step = pl.program_id(0)

    @pl.when(step == 0)
    def _init():
        acc_ref[...] = jnp.zeros_like(acc_ref)
        cnt_ref[...] = jnp.zeros_like(cnt_ref)

    x = _combine(n_ref[...], d_ref[...])
    bids = b_ref[...].reshape(1, CBLK)
    gids = jax.lax.broadcasted_iota(jnp.int32, (NUM_GRAPHS, CBLK), 0)
    mask = (gids == bids).astype(jnp.float32)
    acc_ref[...] += jax.lax.dot_general(
        x, mask, (((1,), (1,)), ((), ())), preferred_element_type=jnp.float32)
    cnt_ref[:1, :] += jnp.sum(mask, axis=1)[None, :]

    @pl.when(step == NCB - 1)
    def _fin():
        mean = acc_ref[...] / jnp.maximum(cnt_ref[:1, :], 1.0)
        row = jax.lax.broadcasted_iota(jnp.int32, (32, NUM_GRAPHS), 0)
        z = jnp.where(row < 10, mean, -jnp.inf)
        m = jnp.max(z, axis=0, keepdims=True)
        e = jnp.exp(z - m)
        sm = e / jnp.sum(e, axis=0, keepdims=True)
        o_ref[...] = jnp.concatenate(
            [sm, jnp.zeros((32, 128 - NUM_GRAPHS), jnp.float32)], axis=1)


def _proj0(x1t, wcat_t, amat):
    return pl.pallas_call(
        _proj0_body,
        grid=(NCB,),
        in_specs=[
            pl.BlockSpec((128, CBLK), lambda i: (0, i)),
            pl.BlockSpec((96, 128), lambda i: (0, 0)),
            pl.BlockSpec((8, 96), lambda i: (0, 0)),
        ],
        out_specs=[
            pl.BlockSpec((96, CBLK), lambda i: (0, i)),
            pl.BlockSpec((8, CBLK), lambda i: (0, i)),
        ],
        out_shape=[
            jax.ShapeDtypeStruct((96, NPAD), jnp.float32),
            jax.ShapeDtypeStruct((8, NPAD), jnp.float32),
        ],
    )(x1t, wcat_t, amat)


def _projc(nums, dens, wcat_t, amat):
    return pl.pallas_call(
        _projc_body,
        grid=(NCB,),
        in_specs=[
            pl.BlockSpec((2, 3, 32, CBLK), lambda i: (0, 0, 0, i)),
            pl.BlockSpec((8, CBLK), lambda i: (0, i)),
            pl.BlockSpec((96, 32), lambda i: (0, 0)),
            pl.BlockSpec((8, 96), lambda i: (0, 0)),
        ],
        out_specs=[
            pl.BlockSpec((96, CBLK), lambda i: (0, i)),
            pl.BlockSpec((8, CBLK), lambda i: (0, i)),
        ],
        out_shape=[
            jax.ShapeDtypeStruct((96, NPAD), jnp.float32),
            jax.ShapeDtypeStruct((8, NPAD), jnp.float32),
        ],
    )(nums, dens, wcat_t, amat)


def _pool(nums, dens, batchp):
    return pl.pallas_call(
        _pool_body,
        grid=(NCB,),
        in_specs=[
            pl.BlockSpec((2, 3, 32, CBLK), lambda i: (0, 0, 0, i)),
            pl.BlockSpec((8, CBLK), lambda i: (0, i)),
            pl.BlockSpec((1, 1, CBLK), lambda i: (i, 0, 0)),
        ],
        out_specs=pl.BlockSpec((32, 128), lambda i: (0, 0)),
        out_shape=jax.ShapeDtypeStruct((32, 128), jnp.float32),
        scratch_shapes=[
            pltpu.VMEM((32, NUM_GRAPHS), jnp.float32),
            pltpu.VMEM((8, NUM_GRAPHS), jnp.float32),
        ],
    )(nums, dens, batchp)


_MESH = plsc.VectorSubcoreMesh(core_axis_name="c", subcore_axis_name="s")
HALF = NNZ // 2
PTILE = HALF // 16
SEG = NPAD // 16


def _zero(ref, base, n):
    @plsc.parallel_loop(0, n // 16, unroll=8)
    def body(k):
        ref[pl.ds(base + k * 16, 16)] = jnp.zeros((16,), jnp.float32)


def _unpack(pk):
    return jnp.right_shift(pk, 14), jnp.bitwise_and(pk, 16383)


def _sc_edges_body(last, htf, smf, l1pk, l1v, lupk, ldpk, nums, dens,
                   hbuf0, hbuf1, acc0, acc1, pbuf, wbuf, pbuf2, wbuf2,
                   mbuf, w_sh, slab, sem_a, sem_b):
    c = lax.axis_index("c")
    s = lax.axis_index("s")

    for g, epk in ((0, lupk), (1, ldpk)):
        pltpu.sync_copy(smf.at[pl.ds((2 * g) * NPAD, NPAD)], hbuf0)
        pltpu.sync_copy(smf.at[pl.ds((2 * g + 1) * NPAD, NPAD)], hbuf1)
        _zero(acc0, 0, NPAD)
        base0 = c * HALF + s * PTILE

        for cbase, clen in ((0, EC), (EC, PTILE - EC)):
            pltpu.sync_copy(epk.at[pl.ds(base0 + cbase, clen)],
                            pbuf.at[pl.ds(0, clen)])

            @plsc.parallel_loop(0, clen // 16, unroll=5)
            def grp0(q):
                off = q * 16
                i16, j16 = _unpack(pbuf[pl.ds(off, 16)])
                sd = plsc.load_gather(hbuf0, [i16])
                ss = plsc.load_gather(hbuf1, [j16])
                e = sd + ss
                e = jnp.maximum(e, 0.2 * e)
                w = jnp.exp(e)
                wbuf[pl.ds(off, 16)] = w
                plsc.addupdate_scatter(acc0, [i16], w)
            pltpu.sync_copy(
                wbuf.at[pl.ds(0, clen)],
                w_sh.at[pl.ds(g * NNZ + base0 + cbase, clen)])

        pltpu.sync_copy(acc0, slab.at[s])
        plsc.subcore_barrier()

        pltpu.sync_copy(slab.at[:, pl.ds(s * SEG, SEG)], mbuf)

        @plsc.parallel_loop(0, SEG // 16, unroll=2)
        def red(q):
            v = mbuf[0, pl.ds(q * 16, 16)]
            for t in range(1, 16):
                v = v + mbuf[t, pl.ds(q * 16, 16)]
            acc1[pl.ds(q * 16, 16)] = v
        pltpu.sync_copy(acc1.at[pl.ds(0, SEG)],
                        dens.at[pl.ds((2 * c + g) * NPAD + s * SEG, SEG)])
        if g == 0:
            _zero(acc0, 0, SEG)
            for r in range(4, 8):
                pltpu.sync_copy(acc0.at[pl.ds(0, SEG)],
                                dens.at[pl.ds(r * NPAD + s * SEG, SEG)])
        plsc.subcore_barrier()

    nchunk = HALF // EC

    def _phase(m, epk):
        f0 = 32 * m + 2 * s
        pltpu.sync_copy(htf.at[pl.ds(f0 * NPAD, NPAD)], hbuf0)
        pltpu.sync_copy(htf.at[pl.ds((f0 + 1) * NPAD, NPAD)], hbuf1)
        _zero(acc0, 0, NPAD)
        _zero(acc1, 0, NPAD)
        ebase = c * HALF

        def start_w(kc, wb, sem):
            if m == 0:
                return pltpu.async_copy(
                    l1v.at[pl.ds(ebase + kc * EC, EC)], wb, sem)
            return pltpu.async_copy(
                w_sh.at[pl.ds((m - 1) * NNZ + ebase + kc * EC, EC)], wb, sem)

        def compute(pb, wb):
            @plsc.parallel_loop(0, EC // 16, unroll=20)
            def grp(q):
                off = q * 16
                i16, j16 = _unpack(pb[pl.ds(off, 16)])
                w16 = wb[pl.ds(off, 16)]
                h0 = plsc.load_gather(hbuf0, [j16])
                plsc.addupdate_scatter(acc0, [i16], w16 * h0)
                h1 = plsc.load_gather(hbuf1, [j16])
                plsc.addupdate_scatter(acc1, [i16], w16 * h1)

        def pair(k, _):
            ha = pltpu.async_copy(
                epk.at[pl.ds(ebase + 2 * k * EC, EC)], pbuf, sem_a)
            hwa = start_w(2 * k, wbuf, sem_a)
            hb = pltpu.async_copy(
                epk.at[pl.ds(ebase + (2 * k + 1) * EC, EC)], pbuf2, sem_b)
            hwb = start_w(2 * k + 1, wbuf2, sem_b)
            ha.wait()
            hwa.wait()
            compute(pbuf, wbuf)
            hb.wait()
            hwb.wait()
            compute(pbuf2, wbuf2)
            return 0
        lax.fori_loop(0, nchunk // 2, pair, 0)

        row = (c * 3 + m) * 32 + 2 * s
        pltpu.sync_copy(acc0, nums.at[pl.ds(row * NPAD, NPAD)])
        pltpu.sync_copy(acc1, nums.at[pl.ds((row + 1) * NPAD, NPAD)])

    for m, epk in ((0, l1pk), (1, lupk), (2, ldpk)):
        if last:
            @pl.when(s < 5)
            def _active():
                _phase(m, epk)
        else:
            _phase(m, epk)


def _make_sc(last):
    return functools.partial(
        pl.kernel,
        mesh=_MESH,
        compiler_params=pltpu.CompilerParams(needs_layout_passes=False),
        out_type=[
            jax.ShapeDtypeStruct((2 * 3 * 32 * NPAD,), jnp.float32),
            jax.ShapeDtypeStruct((8 * NPAD,), jnp.float32),
        ],
        scratch_types=[
            pltpu.VMEM((NPAD,), jnp.float32),
            pltpu.VMEM((NPAD,), jnp.float32),
            pltpu.VMEM((NPAD,), jnp.float32),
            pltpu.VMEM((NPAD,), jnp.float32),
            pltpu.VMEM((EC,), jnp.int32),
            pltpu.VMEM((EC,), jnp.float32),
            pltpu.VMEM((EC,), jnp.int32),
            pltpu.VMEM((EC,), jnp.float32),
            pltpu.VMEM((16, SEG), jnp.float32),
            pltpu.HBM((2 * NNZ,), jnp.float32),
            pltpu.VMEM_SHARED((16, NPAD), jnp.float32),
            pltpu.SemaphoreType.DMA,
            pltpu.SemaphoreType.DMA,
        ],
    )(functools.partial(_sc_edges_body, last))


_sc_edges = _make_sc(False)
_sc_edges_last = _make_sc(True)


def _prep_params(params):
    out = []
    for p in params:
        din = p['Wp'].shape[0]
        dout = p['Wp'].shape[1]
        wp = jnp.zeros((din, 32), jnp.float32).at[:, :dout].set(p['Wp'])
        wu = jnp.zeros((din, 32), jnp.float32).at[:, :dout].set(p['Wu'])
        wd = jnp.zeros((din, 32), jnp.float32).at[:, :dout].set(p['Wd'])
        wcat_t = jnp.concatenate([wp.T, wu.T, wd.T], axis=0)
        amat = jnp.zeros((8, 96), jnp.float32)
        amat = amat.at[0, 32:32 + dout].set(p['au_d'])
        amat = amat.at[1, 32:32 + dout].set(p['au_s'])
        amat = amat.at[2, 64:64 + dout].set(p['ad_d'])
        amat = amat.at[3, 64:64 + dout].set(p['ad_s'])
        out.append((wcat_t, amat))
    return out


def kernel(X1, L1_idx, L1_val, Lu_idx, Lu_val, Ld_idx, Ld_val, batch1, params):
    pp = _prep_params(params)
    x1t = jnp.zeros((128, NPAD), jnp.float32).at[:, :N].set(X1.T)
    batchp = jnp.full((NPAD,), NUM_GRAPHS + 1, jnp.int32).at[:N].set(batch1)
    batchp = batchp.reshape(NCB, 1, CBLK)
    l1pk = jnp.left_shift(L1_idx[0], 14) | L1_idx[1]
    lupk = jnp.left_shift(Lu_idx[0], 14) | Lu_idx[1]
    ldpk = jnp.left_shift(Ld_idx[0], 14) | Ld_idx[1]

    ht, smat = _proj0(x1t, pp[0][0], pp[0][1])
    for l in range(1, 5):
        sc = _sc_edges_last if l == 4 else _sc_edges
        numsf, densf = sc(ht.reshape(-1), smat.reshape(-1),
                          l1pk, L1_val, lupk, ldpk)
        nums = numsf.reshape(2, 3, 32, NPAD)
        dens = densf.reshape(8, NPAD)
        if l < 4:
            ht, smat = _projc(nums, dens, pp[l][0], pp[l][1])
    out = _pool(nums, dens, batchp)
    return out[:10, :NUM_GRAPHS].T


# speedup vs baseline: 1.0230x; 1.0066x over previous
import functools
import jax
import jax.numpy as jnp
from jax import lax
from jax.experimental import pallas as pl
from jax.experimental.pallas import tpu as pltpu
from jax.experimental.pallas import tpu_sc as plsc

N = 10000
NNZ = 320000
NUM_GRAPHS = 64
NPAD = 10240
CBLK = 512
NCB = NPAD // CBLK
EC = 8000
EPS = 1e-16


def _proj0_body(x_ref, w_ref, a_ref, h_ref, s_ref):
    h = jax.lax.dot(w_ref[...], x_ref[...], preferred_element_type=jnp.float32)
    h_ref[...] = h
    s_ref[...] = jax.lax.dot(a_ref[...], h, preferred_element_type=jnp.float32)


def _combine(nums, dens):
    nu = nums[0, 1] + nums[1, 1]
    nd = nums[0, 2] + nums[1, 2]
    np_ = nums[0, 0] + nums[1, 0]
    du = (dens[0] + dens[2])[None, :]
    dd = (dens[1] + dens[3])[None, :]
    return jax.nn.relu(nu / (du + EPS) + nd / (dd + EPS) + np_)


def _projc_body(n_ref, d_ref, w_ref, a_ref, h_ref, s_ref):
    x = _combine(n_ref[...], d_ref[...])
    h = jax.lax.dot(w_ref[...], x, preferred_element_type=jnp.float32)
    h_ref[...] = h
    s_ref[...] = jax.lax.dot(a_ref[...], h, preferred_element_type=jnp.float32)


def _pool_body(n_ref, d_ref, b_ref, o_ref, acc_ref, cnt_ref):
    step = pl.program_id(0)

    @pl.when(step == 0)
    def _init():
        acc_ref[...] = jnp.zeros_like(acc_ref)
        cnt_ref[...] = jnp.zeros_like(cnt_ref)

    nums = n_ref[...]
    n = nums[:, :, :16, :] + nums[:, :, 16:, :]
    x = _combine(n, d_ref[...])
    bids = b_ref[...].reshape(1, CBLK)
    gids = jax.lax.broadcasted_iota(jnp.int32, (NUM_GRAPHS, CBLK), 0)
    mask = (gids == bids).astype(jnp.float32)
    acc_ref[...] += jax.lax.dot_general(
        x, mask, (((1,), (1,)), ((), ())), preferred_element_type=jnp.float32)
    cnt_ref[:1, :] += jnp.sum(mask, axis=1)[None, :]

    @pl.when(step == NCB - 1)
    def _fin():
        mean = acc_ref[...] / jnp.maximum(cnt_ref[:1, :], 1.0)
        row = jax.lax.broadcasted_iota(jnp.int32, (16, NUM_GRAPHS), 0)
        z = jnp.where(row < 10, mean, -jnp.inf)
        m = jnp.max(z, axis=0, keepdims=True)
        e = jnp.exp(z - m)
        sm = e / jnp.sum(e, axis=0, keepdims=True)
        o_ref[...] = jnp.concatenate(
            [sm, jnp.zeros((16, 128 - NUM_GRAPHS), jnp.float32)], axis=1)


def _proj0(x1t, wcat_t, amat):
    return pl.pallas_call(
        _proj0_body,
        grid=(NCB,),
        in_specs=[
            pl.BlockSpec((128, CBLK), lambda i: (0, i)),
            pl.BlockSpec((96, 128), lambda i: (0, 0)),
            pl.BlockSpec((8, 96), lambda i: (0, 0)),
        ],
        out_specs=[
            pl.BlockSpec((96, CBLK), lambda i: (0, i)),
            pl.BlockSpec((8, CBLK), lambda i: (0, i)),
        ],
        out_shape=[
            jax.ShapeDtypeStruct((96, NPAD), jnp.float32),
            jax.ShapeDtypeStruct((8, NPAD), jnp.float32),
        ],
    )(x1t, wcat_t, amat)


def _projc(nums, dens, wcat_t, amat):
    return pl.pallas_call(
        _projc_body,
        grid=(NCB,),
        in_specs=[
            pl.BlockSpec((2, 3, 32, CBLK), lambda i: (0, 0, 0, i)),
            pl.BlockSpec((8, CBLK), lambda i: (0, i)),
            pl.BlockSpec((96, 32), lambda i: (0, 0)),
            pl.BlockSpec((8, 96), lambda i: (0, 0)),
        ],
        out_specs=[
            pl.BlockSpec((96, CBLK), lambda i: (0, i)),
            pl.BlockSpec((8, CBLK), lambda i: (0, i)),
        ],
        out_shape=[
            jax.ShapeDtypeStruct((96, NPAD), jnp.float32),
            jax.ShapeDtypeStruct((8, NPAD), jnp.float32),
        ],
    )(nums, dens, wcat_t, amat)


def _pool(nums, dens, batchp):
    return pl.pallas_call(
        _pool_body,
        grid=(NCB,),
        in_specs=[
            pl.BlockSpec((2, 3, 32, CBLK), lambda i: (0, 0, 0, i)),
            pl.BlockSpec((8, CBLK), lambda i: (0, i)),
            pl.BlockSpec((1, 1, CBLK), lambda i: (i, 0, 0)),
        ],
        out_specs=pl.BlockSpec((16, 128), lambda i: (0, 0)),
        out_shape=jax.ShapeDtypeStruct((16, 128), jnp.float32),
        scratch_shapes=[
            pltpu.VMEM((16, NUM_GRAPHS), jnp.float32),
            pltpu.VMEM((8, NUM_GRAPHS), jnp.float32),
        ],
    )(nums, dens, batchp)


_MESH = plsc.VectorSubcoreMesh(core_axis_name="c", subcore_axis_name="s")
HALF = NNZ // 2
PTILE = HALF // 16
SEG = NPAD // 16


def _zero(ref, base, n):
    @plsc.parallel_loop(0, n // 16, unroll=8)
    def body(k):
        ref[pl.ds(base + k * 16, 16)] = jnp.zeros((16,), jnp.float32)


def _unpack(pk):
    return jnp.right_shift(pk, 14), jnp.bitwise_and(pk, 16383)


def _sc_edges_body(last, htf, smf, l1pk, l1v, lupk, ldpk, nums, dens,
                   hbuf0, hbuf1, acc0, acc1, pbuf, wbuf, pbuf2, wbuf2,
                   mbuf, w_sh, slab, sem_a, sem_b):
    c = lax.axis_index("c")
    s = lax.axis_index("s")

    for g, epk in ((0, lupk), (1, ldpk)):
        pltpu.sync_copy(smf.at[pl.ds((2 * g) * NPAD, NPAD)], hbuf0)
        pltpu.sync_copy(smf.at[pl.ds((2 * g + 1) * NPAD, NPAD)], hbuf1)
        _zero(acc0, 0, NPAD)
        base0 = c * HALF + s * PTILE

        for cbase, clen in ((0, EC), (EC, PTILE - EC)):
            pltpu.sync_copy(epk.at[pl.ds(base0 + cbase, clen)],
                            pbuf.at[pl.ds(0, clen)])

            @plsc.parallel_loop(0, clen // 16, unroll=5)
            def grp0(q):
                off = q * 16
                i16, j16 = _unpack(pbuf[pl.ds(off, 16)])
                sd = plsc.load_gather(hbuf0, [i16])
                ss = plsc.load_gather(hbuf1, [j16])
                e = sd + ss
                e = jnp.maximum(e, 0.2 * e)
                w = jnp.exp(e)
                wbuf[pl.ds(off, 16)] = w
                plsc.addupdate_scatter(acc0, [i16], w)
            pltpu.sync_copy(
                wbuf.at[pl.ds(0, clen)],
                w_sh.at[pl.ds(g * NNZ + base0 + cbase, clen)])

        pltpu.sync_copy(acc0, slab.at[s])
        plsc.subcore_barrier()

        pltpu.sync_copy(slab.at[:, pl.ds(s * SEG, SEG)], mbuf)

        @plsc.parallel_loop(0, SEG // 16, unroll=2)
        def red(q):
            v = mbuf[0, pl.ds(q * 16, 16)]
            for t in range(1, 16):
                v = v + mbuf[t, pl.ds(q * 16, 16)]
            acc1[pl.ds(q * 16, 16)] = v
        pltpu.sync_copy(acc1.at[pl.ds(0, SEG)],
                        dens.at[pl.ds((2 * c + g) * NPAD + s * SEG, SEG)])
        if g == 0:
            _zero(acc0, 0, SEG)
            for r in range(4, 8):
                pltpu.sync_copy(acc0.at[pl.ds(0, SEG)],
                                dens.at[pl.ds(r * NPAD + s * SEG, SEG)])
        plsc.subcore_barrier()

    nchunk = HALF // EC

    def _phase(m, epk):
        f0 = 32 * m + 2 * s
        pltpu.sync_copy(htf.at[pl.ds(f0 * NPAD, NPAD)], hbuf0)
        pltpu.sync_copy(htf.at[pl.ds((f0 + 1) * NPAD, NPAD)], hbuf1)
        _zero(acc0, 0, NPAD)
        _zero(acc1, 0, NPAD)
        ebase = c * HALF

        def start_w(kc, wb, sem):
            if m == 0:
                return pltpu.async_copy(
                    l1v.at[pl.ds(ebase + kc * EC, EC)], wb, sem)
            return pltpu.async_copy(
                w_sh.at[pl.ds((m - 1) * NNZ + ebase + kc * EC, EC)], wb, sem)

        def compute(pb, wb):
            @plsc.parallel_loop(0, EC // 16, unroll=20)
            def grp(q):
                off = q * 16
                i16, j16 = _unpack(pb[pl.ds(off, 16)])
                w16 = wb[pl.ds(off, 16)]
                h0 = plsc.load_gather(hbuf0, [j16])
                plsc.addupdate_scatter(acc0, [i16], w16 * h0)
                h1 = plsc.load_gather(hbuf1, [j16])
                plsc.addupdate_scatter(acc1, [i16], w16 * h1)

        def pair(k, _):
            ha = pltpu.async_copy(
                epk.at[pl.ds(ebase + 2 * k * EC, EC)], pbuf, sem_a)
            hwa = start_w(2 * k, wbuf, sem_a)
            hb = pltpu.async_copy(
                epk.at[pl.ds(ebase + (2 * k + 1) * EC, EC)], pbuf2, sem_b)
            hwb = start_w(2 * k + 1, wbuf2, sem_b)
            ha.wait()
            hwa.wait()
            compute(pbuf, wbuf)
            hb.wait()
            hwb.wait()
            compute(pbuf2, wbuf2)
            return 0
        lax.fori_loop(0, nchunk // 2, pair, 0)

        row = (c * 3 + m) * 32 + 2 * s
        pltpu.sync_copy(acc0, nums.at[pl.ds(row * NPAD, NPAD)])
        pltpu.sync_copy(acc1, nums.at[pl.ds((row + 1) * NPAD, NPAD)])

    def _phase_last(m, epk):
        fsel = jnp.where(s < 10, s, s - 10)
        row_off = jnp.where(s < 10, s, s + 6)
        clo = jnp.where(s >= 10, nchunk // 2, 0)
        chi = jnp.where(s < 6, nchunk // 2, nchunk)
        pltpu.sync_copy(htf.at[pl.ds((32 * m + fsel) * NPAD, NPAD)], hbuf0)
        _zero(acc0, 0, NPAD)
        ebase = c * HALF

        def chunk(k, _):
            pltpu.sync_copy(epk.at[pl.ds(ebase + k * EC, EC)], pbuf)
            if m == 0:
                pltpu.sync_copy(l1v.at[pl.ds(ebase + k * EC, EC)], wbuf)
            else:
                pltpu.sync_copy(
                    w_sh.at[pl.ds((m - 1) * NNZ + ebase + k * EC, EC)], wbuf)

            @plsc.parallel_loop(0, EC // 16, unroll=20)
            def grp(q):
                off = q * 16
                i16, j16 = _unpack(pbuf[pl.ds(off, 16)])
                w16 = wbuf[pl.ds(off, 16)]
                h0 = plsc.load_gather(hbuf0, [j16])
                plsc.addupdate_scatter(acc0, [i16], w16 * h0)
            return 0
        lax.fori_loop(clo, chi, chunk, 0)

        row = (c * 3 + m) * 32 + row_off
        pltpu.sync_copy(acc0, nums.at[pl.ds(row * NPAD, NPAD)])

        @pl.when((s >= 10) & (s < 14))
        def _zrows():
            _zero(acc1, 0, NPAD)
            zrow = (c * 3 + m) * 32 + 12 + s
            pltpu.sync_copy(acc1, nums.at[pl.ds(zrow * NPAD, NPAD)])

    for m, epk in ((0, l1pk), (1, lupk), (2, ldpk)):
        if last:
            _phase_last(m, epk)
        else:
            _phase(m, epk)


def _make_sc(last):
    return functools.partial(
        pl.kernel,
        mesh=_MESH,
        compiler_params=pltpu.CompilerParams(needs_layout_passes=False),
        out_type=[
            jax.ShapeDtypeStruct((2 * 3 * 32 * NPAD,), jnp.float32),
            jax.ShapeDtypeStruct((8 * NPAD,), jnp.float32),
        ],
        scratch_types=[
            pltpu.VMEM((NPAD,), jnp.float32),
            pltpu.VMEM((NPAD,), jnp.float32),
            pltpu.VMEM((NPAD,), jnp.float32),
            pltpu.VMEM((NPAD,), jnp.float32),
            pltpu.VMEM((EC,), jnp.int32),
            pltpu.VMEM((EC,), jnp.float32),
            pltpu.VMEM((EC,), jnp.int32),
            pltpu.VMEM((EC,), jnp.float32),
            pltpu.VMEM((16, SEG), jnp.float32),
            pltpu.HBM((2 * NNZ,), jnp.float32),
            pltpu.VMEM_SHARED((16, NPAD), jnp.float32),
            pltpu.SemaphoreType.DMA,
            pltpu.SemaphoreType.DMA,
        ],
    )(functools.partial(_sc_edges_body, last))


_sc_edges = _make_sc(False)
_sc_edges_last = _make_sc(True)


def _prep_params(params):
    out = []
    for p in params:
        din = p['Wp'].shape[0]
        dout = p['Wp'].shape[1]
        wp = jnp.zeros((din, 32), jnp.float32).at[:, :dout].set(p['Wp'])
        wu = jnp.zeros((din, 32), jnp.float32).at[:, :dout].set(p['Wu'])
        wd = jnp.zeros((din, 32), jnp.float32).at[:, :dout].set(p['Wd'])
        wcat_t = jnp.concatenate([wp.T, wu.T, wd.T], axis=0)
        amat = jnp.zeros((8, 96), jnp.float32)
        amat = amat.at[0, 32:32 + dout].set(p['au_d'])
        amat = amat.at[1, 32:32 + dout].set(p['au_s'])
        amat = amat.at[2, 64:64 + dout].set(p['ad_d'])
        amat = amat.at[3, 64:64 + dout].set(p['ad_s'])
        out.append((wcat_t, amat))
    return out


def kernel(X1, L1_idx, L1_val, Lu_idx, Lu_val, Ld_idx, Ld_val, batch1, params):
    pp = _prep_params(params)
    x1t = jnp.zeros((128, NPAD), jnp.float32).at[:, :N].set(X1.T)
    batchp = jnp.full((NPAD,), NUM_GRAPHS + 1, jnp.int32).at[:N].set(batch1)
    batchp = batchp.reshape(NCB, 1, CBLK)
    l1pk = jnp.left_shift(L1_idx[0], 14) | L1_idx[1]
    lupk = jnp.left_shift(Lu_idx[0], 14) | Lu_idx[1]
    ldpk = jnp.left_shift(Ld_idx[0], 14) | Ld_idx[1]

    ht, smat = _proj0(x1t, pp[0][0], pp[0][1])
    for l in range(1, 5):
        sc = _sc_edges_last if l == 4 else _sc_edges
        numsf, densf = sc(ht.reshape(-1), smat.reshape(-1),
                          l1pk, L1_val, lupk, ldpk)
        nums = numsf.reshape(2, 3, 32, NPAD)
        dens = densf.reshape(8, NPAD)
        if l < 4:
            ht, smat = _projc(nums, dens, pp[l][0], pp[l][1])
    out = _pool(nums, dens, batchp)
    return out[:10, :NUM_GRAPHS].T
